# baseline (speedup 1.0000x reference)
import functools

import jax
import jax.numpy as jnp
from jax import lax
from jax.experimental import pallas as pl
from jax.experimental.pallas import tpu as pltpu
from jax.experimental.pallas import tpu_sc as plsc

WINDOW = 127
LANES = 16
NCHUNK = 8
XPAD = 144
NBUF = 3
B_SC = 2

YPAD = 136
XTILE = 256
CB = 8


def _sc_body(C, W, pairs_per_worker, num_cores,
             in_hbm, pos_hbm, out_hbm, pos_v, slab, stage, in_sem, out_sem):
    wid = lax.axis_index("s") * num_cores + lax.axis_index("c")
    pltpu.sync_copy(pos_hbm, pos_v)

    def scalar_at(k):
        splat = plsc.load_gather(pos_v, [jnp.full((LANES,), k, jnp.int32)])
        return jnp.max(splat)

    def coords(t):
        pair = wid * pairs_per_worker + t
        return pair // C, pair % C

    def window(t):
        b, c = coords(t)
        y = scalar_at(2 * b)
        x = scalar_at(2 * b + 1)
        xb = pl.multiple_of(
            lax.min((x // LANES) * LANES, jnp.int32(W - XPAD)), LANES)
        return b, c, y, xb, x - xb

    def start_in(t, k):
        b, c, y, xb, _ = window(t)
        pltpu.make_async_copy(
            in_hbm.at[b, c, pl.ds(y, WINDOW), pl.ds(xb, XPAD)],
            slab.at[k], in_sem.at[k]).start()

    def wait_in(k):
        pltpu.make_async_copy(
            in_hbm.at[0, 0, pl.ds(0, WINDOW), pl.ds(0, XPAD)],
            slab.at[k], in_sem.at[k]).wait()

    def start_out(t, k):
        b, c = coords(t)
        pltpu.make_async_copy(
            stage.at[k], out_hbm.at[b, c], out_sem.at[k]).start()

    def wait_out(k):
        pltpu.make_async_copy(
            stage.at[k], out_hbm.at[0, 0], out_sem.at[k]).wait()

    offs = tuple(j * LANES for j in range(NCHUNK - 1)) + (WINDOW - LANES,)

    def shift(t, k):
        _, _, _, _, dx = window(t)

        @plsc.parallel_loop(0, WINDOW, unroll=4)
        def _row(i):
            vals = [slab[k, i, pl.ds(dx + o, LANES)] for o in offs]
            for o, v in zip(offs, vals):
                stage[k, i, pl.ds(o, LANES)] = v

    for p in range(NBUF):
        start_in(p, p)

    def step(t, carry):
        k = lax.rem(t, NBUF)
        wait_in(k)

        @pl.when(t >= NBUF)
        def _():
            wait_out(k)

        shift(t, k)
        start_out(t, k)

        @pl.when(t + NBUF < pairs_per_worker)
        def _():
            start_in(t + NBUF, k)

        return carry

    lax.fori_loop(0, pairs_per_worker, step, 0)
    for p in range(NBUF):
        wait_out(p)


def _sc_kernel(input, pos32):
    B, C, H, W = input.shape
    info = plsc.get_sparse_core_info()
    num_workers = info.num_cores * info.num_subcores
    pairs_per_worker = (B_SC * C) // num_workers
    mesh = plsc.VectorSubcoreMesh(core_axis_name="c", subcore_axis_name="s")
    run = pl.kernel(
        functools.partial(_sc_body, C, W, pairs_per_worker, info.num_cores),
        out_type=jax.ShapeDtypeStruct((B_SC, C, WINDOW, WINDOW), input.dtype),
        mesh=mesh,
        scratch_types=[
            pltpu.VMEM((2 * B_SC,), jnp.int32),
            pltpu.VMEM((NBUF, WINDOW, XPAD), jnp.float32),
            pltpu.VMEM((NBUF, WINDOW, WINDOW), jnp.float32),
            pltpu.SemaphoreType.DMA((NBUF,)),
            pltpu.SemaphoreType.DMA((NBUF,)),
        ],
        compiler_params=pltpu.CompilerParams(
            use_tc_tiling_on_sc=False, needs_layout_passes=False),
    )
    return run(input, pos32[:B_SC].reshape(-1))


def _tc_body(pos_ref, in_ref, out_ref):
    b = pl.program_id(0) + B_SC
    y = pos_ref[b, 0]
    x = pos_ref[b, 1]
    dy = y - jnp.minimum((y // 8) * 8, 512 - YPAD)
    dx = x - jnp.minimum((x // 128) * 128, 512 - XTILE)
    blk = in_ref[0]
    blk = pltpu.roll(blk, YPAD - dy, 1)
    blk = pltpu.roll(blk, XTILE - dx, 2)
    out_ref[0] = blk[:, :WINDOW, :WINDOW]


def _tc_kernel(input, pos32):
    B, C, H, W = input.shape

    def in_map(b, c, pos_ref):
        ymin = pl.multiple_of(
            jnp.minimum((pos_ref[b + B_SC, 0] // 8) * 8, 512 - YPAD), 8)
        xmin = pl.multiple_of(
            jnp.minimum((pos_ref[b + B_SC, 1] // 128) * 128, 512 - XTILE),
            128)
        return b + B_SC, c * CB, ymin, xmin

    return pl.pallas_call(
        _tc_body,
        grid_spec=pltpu.PrefetchScalarGridSpec(
            num_scalar_prefetch=1,
            grid=(B - B_SC, C // CB),
            in_specs=[
                pl.BlockSpec(
                    (pl.Element(1), pl.Element(CB), pl.Element(YPAD),
                     pl.Element(XTILE)),
                    in_map,
                )
            ],
            out_specs=pl.BlockSpec(
                (pl.Element(1), pl.Element(CB), pl.Element(WINDOW),
                 pl.Element(WINDOW)),
                lambda b, c, pos_ref: (b, c * CB, 0, 0),
            ),
        ),
        out_shape=jax.ShapeDtypeStruct(
            (B - B_SC, C, WINDOW, WINDOW), input.dtype),
    )(pos32, input)


def kernel(input, pos):
    pos32 = pos.astype(jnp.int32)
    out_sc = _sc_kernel(input, pos32)
    out_tc = _tc_kernel(input, pos32)
    return jnp.concatenate([out_sc, out_tc], axis=0)


# speedup vs baseline: 1.0757x; 1.0757x over previous
import functools

import jax
import jax.numpy as jnp
from jax import lax
from jax.experimental import pallas as pl
from jax.experimental.pallas import tpu as pltpu
from jax.experimental.pallas import tpu_sc as plsc

WINDOW = 127
LANES = 16
NCHUNK = 8
XPAD = 144
NBUF = 3


def _sc_body(C, W, pairs_per_worker, num_cores,
             in_hbm, pos_hbm, out_hbm, pos_v, slab, stage, in_sem, out_sem):
    wid = lax.axis_index("s") * num_cores + lax.axis_index("c")
    pltpu.sync_copy(pos_hbm, pos_v)

    def scalar_at(k):
        splat = plsc.load_gather(pos_v, [jnp.full((LANES,), k, jnp.int32)])
        return jnp.max(splat)

    def coords(t):
        pair = wid * pairs_per_worker + t
        return pair // C, pair % C

    def window(t):
        b, c = coords(t)
        y = scalar_at(2 * b)
        x = scalar_at(2 * b + 1)
        xb = pl.multiple_of(
            lax.min((x // LANES) * LANES, jnp.int32(W - XPAD)), LANES)
        return b, c, y, xb, x - xb

    def start_in(t, k):
        b, c, y, xb, _ = window(t)
        pltpu.make_async_copy(
            in_hbm.at[b, c, pl.ds(y, WINDOW), pl.ds(xb, XPAD)],
            slab.at[k], in_sem.at[k]).start()

    def wait_in(k):
        pltpu.make_async_copy(
            in_hbm.at[0, 0, pl.ds(0, WINDOW), pl.ds(0, XPAD)],
            slab.at[k], in_sem.at[k]).wait()

    def start_out(t, k):
        b, c = coords(t)
        pltpu.make_async_copy(
            stage.at[k], out_hbm.at[b, c], out_sem.at[k]).start()

    def wait_out(k):
        pltpu.make_async_copy(
            stage.at[k], out_hbm.at[0, 0], out_sem.at[k]).wait()

    offs = tuple(j * LANES for j in range(NCHUNK - 1)) + (WINDOW - LANES,)

    def shift(t, k):
        _, _, _, _, dx = window(t)

        @plsc.parallel_loop(0, WINDOW, unroll=4)
        def _row(i):
            vals = [slab[k, i, pl.ds(dx + o, LANES)] for o in offs]
            for o, v in zip(offs, vals):
                stage[k, i, pl.ds(o, LANES)] = v

    for p in range(NBUF):
        start_in(p, p)

    def step(t, carry):
        k = lax.rem(t, NBUF)
        wait_in(k)

        @pl.when(t >= NBUF)
        def _():
            wait_out(k)

        shift(t, k)
        start_out(t, k)

        @pl.when(t + NBUF < pairs_per_worker)
        def _():
            start_in(t + NBUF, k)

        return carry

    lax.fori_loop(0, pairs_per_worker, step, 0)
    for p in range(NBUF):
        wait_out(p)


def kernel(input, pos):
    B, C, H, W = input.shape
    info = plsc.get_sparse_core_info()
    num_workers = info.num_cores * info.num_subcores
    pairs_per_worker = (B * C) // num_workers
    mesh = plsc.VectorSubcoreMesh(core_axis_name="c", subcore_axis_name="s")
    run = pl.kernel(
        functools.partial(_sc_body, C, W, pairs_per_worker, info.num_cores),
        out_type=jax.ShapeDtypeStruct((B, C, WINDOW, WINDOW), input.dtype),
        mesh=mesh,
        scratch_types=[
            pltpu.VMEM((2 * B,), jnp.int32),
            pltpu.VMEM((NBUF, WINDOW, XPAD), jnp.float32),
            pltpu.VMEM((NBUF, WINDOW, WINDOW), jnp.float32),
            pltpu.SemaphoreType.DMA((NBUF,)),
            pltpu.SemaphoreType.DMA((NBUF,)),
        ],
        compiler_params=pltpu.CompilerParams(
            use_tc_tiling_on_sc=False, needs_layout_passes=False),
    )
    return run(input, pos.astype(jnp.int32).reshape(-1))
